# baseline (speedup 1.0000x reference)
import functools

import jax
import jax.numpy as jnp
from jax import lax
from jax.experimental import pallas as pl
from jax.experimental.pallas import tpu as pltpu
from jax.experimental.pallas import tpu_sc as plsc

_NC = 2
_NS = 16
_NW = _NC * _NS
_NBUF = 4
_FD = 2
_LANES = 16


@functools.cache
def _build_gather(b_total: int, l_total: int, dim: int):
    assert b_total == _NW * 128 and dim % 8 == 0 and l_total % _NBUF == 0
    n_a = dim // 8
    n_super = l_total // _NBUF
    mesh = plsc.VectorSubcoreMesh(
        core_axis_name="c", subcore_axis_name="s",
        num_cores=_NC, num_subcores=_NS,
    )

    @functools.partial(
        pl.kernel,
        out_type=jax.ShapeDtypeStruct(
            (l_total, n_a, _NW, 8, 128), jnp.float32),
        mesh=mesh,
        scratch_types=[
            pltpu.VMEM((128, l_total), jnp.int32),
            pltpu.VMEM((l_total, 128), jnp.int32),
            pltpu.VMEM((_NBUF, 128, dim), jnp.float32),
            pltpu.VMEM((2, n_a, 8, 128), jnp.float32),
        ]
        + [pltpu.SemaphoreType.DMA] * (_NBUF + 2),
        compiler_params=pltpu.CompilerParams(use_tc_tiling_on_sc=False, needs_layout_passes=False),
    )
    def gather(idx_hbm, table_hbm, out_hbm, idx_v, idx_t, rows_v, tbuf,
               *sems):
        gs = sems[:_NBUF]
        os_ = sems[_NBUF:]
        wid = lax.axis_index("s") * _NC + lax.axis_index("c")

        pltpu.sync_copy(idx_hbm.at[pl.ds(wid * 128, 128)], idx_v)
        lanes = lax.iota(jnp.int32, _LANES)

        @plsc.parallel_loop(0, l_total)
        def idx_t_row(l):
            lvec = jnp.full((_LANES,), 0, jnp.int32) + l
            for k in range(128 // _LANES):
                v = plsc.load_gather(idx_v, [lanes + (k * _LANES), lvec])
                idx_t[l, pl.ds(k * _LANES, _LANES)] = v

        def fire(l, b):
            pltpu.async_copy(table_hbm.at[idx_t.at[l]], rows_v.at[b], gs[b])

        def drain_gather(b):
            pltpu.make_async_copy(
                table_hbm.at[idx_t.at[0]], rows_v.at[b], gs[b]).wait()

        def transpose(b, tb):
            @plsc.parallel_loop(0, 128 // _LANES, unroll=4)
            def krow(k):
                rvec = lanes + k * _LANES
                for d in range(dim):
                    v = plsc.load_gather(
                        rows_v.at[b],
                        [rvec, jnp.full((_LANES,), d, jnp.int32)])
                    tbuf[tb, d // 8, d % 8, pl.ds(k * _LANES, _LANES)] = v

        def out_start(l, tb):
            pltpu.async_copy(tbuf.at[tb], out_hbm.at[l, :, wid], os_[tb])

        def out_wait(tb):
            pltpu.make_async_copy(
                tbuf.at[tb], out_hbm.at[0, :, wid], os_[tb]).wait()

        for lp in range(_FD):
            fire(lp, lp)

        def super_iter(s, carry):
            for b in range(_NBUF):
                l = s * _NBUF + b
                bw = (b + _FD) % _NBUF
                tb = b % 2
                if b + _FD < _NBUF:
                    fire(l + _FD, bw)
                else:
                    @pl.when(s < n_super - 1)
                    def _():
                        fire(l + _FD, bw)
                drain_gather(b)
                if b + _FD < _NBUF:
                    @pl.when(s >= 1)
                    def _():
                        out_wait(tb)
                else:
                    out_wait(tb)
                transpose(b, tb)
                out_start(l, tb)
            return carry

        lax.fori_loop(0, n_super, super_iter, 0)

        for tb in range(2):
            out_wait(tb)

    return gather


def kernel(item_seq, item_emb_weight):
    b, l = item_seq.shape
    dim = item_emb_weight.shape[1]
    idx = item_seq.astype(jnp.int32)
    k = _build_gather(b, l, dim)(idx, item_emb_weight)
    return jnp.transpose(k, (2, 4, 0, 1, 3)).reshape(b, l, dim)


# speedup vs baseline: 1.1335x; 1.1335x over previous
import functools

import jax
import jax.numpy as jnp
from jax import lax
from jax.experimental import pallas as pl
from jax.experimental.pallas import tpu as pltpu
from jax.experimental.pallas import tpu_sc as plsc

_NC = 2
_NS = 16
_NW = _NC * _NS
_NBUF = 4
_FD = 2
_LANES = 16


@functools.cache
def _build_gather(b_total: int, l_total: int, dim: int):
    assert b_total == _NW * 128 and dim % 8 == 0 and l_total % _NBUF == 0
    n_a = dim // 8
    n_super = l_total // _NBUF
    mesh = plsc.VectorSubcoreMesh(
        core_axis_name="c", subcore_axis_name="s",
        num_cores=_NC, num_subcores=_NS,
    )

    @functools.partial(
        pl.kernel,
        out_type=jax.ShapeDtypeStruct(
            (l_total, n_a, _NW, 1024), jnp.float32),
        mesh=mesh,
        scratch_types=[
            pltpu.VMEM((128, l_total), jnp.int32),
            pltpu.VMEM((l_total, 128), jnp.int32),
            pltpu.VMEM((_NBUF, 128, dim), jnp.float32),
            pltpu.VMEM((2, n_a, 1024), jnp.float32),
        ]
        + [pltpu.SemaphoreType.DMA] * (_NBUF + 2),
        compiler_params=pltpu.CompilerParams(use_tc_tiling_on_sc=False, needs_layout_passes=False),
    )
    def gather(idx_hbm, table_hbm, out_hbm, idx_v, idx_t, rows_v, tbuf,
               *sems):
        gs = sems[:_NBUF]
        os_ = sems[_NBUF:]
        wid = lax.axis_index("s") * _NC + lax.axis_index("c")

        pltpu.sync_copy(idx_hbm.at[pl.ds(wid * 128, 128)], idx_v)
        lanes = lax.iota(jnp.int32, _LANES)

        @plsc.parallel_loop(0, l_total)
        def idx_t_row(l):
            lvec = jnp.full((_LANES,), 0, jnp.int32) + l
            for k in range(128 // _LANES):
                v = plsc.load_gather(idx_v, [lanes + (k * _LANES), lvec])
                idx_t[l, pl.ds(k * _LANES, _LANES)] = v

        a_of = []
        rc_base = []
        for j in range(dim // _LANES):
            d_vec = lanes + j * _LANES
            a_of.append(d_vec // 8)
            rc_base.append((d_vec % 8) * 128)

        def fire(l, b):
            pltpu.async_copy(table_hbm.at[idx_t.at[l]], rows_v.at[b], gs[b])

        def drain_gather(b):
            pltpu.make_async_copy(
                table_hbm.at[idx_t.at[0]], rows_v.at[b], gs[b]).wait()

        def transpose(b, tb):
            @plsc.parallel_loop(0, 128)
            def crow(c):
                for j in range(dim // _LANES):
                    v = rows_v[b, c, pl.ds(j * _LANES, _LANES)]
                    plsc.store_scatter(
                        tbuf.at[tb], [a_of[j], rc_base[j] + c], v)

        def out_start(l, tb):
            pltpu.async_copy(tbuf.at[tb], out_hbm.at[l, :, wid], os_[tb])

        def out_wait(tb):
            pltpu.make_async_copy(
                tbuf.at[tb], out_hbm.at[0, :, wid], os_[tb]).wait()

        for lp in range(_FD):
            fire(lp, lp)

        def super_iter(s, carry):
            for b in range(_NBUF):
                l = s * _NBUF + b
                bw = (b + _FD) % _NBUF
                tb = b % 2
                if b + _FD < _NBUF:
                    fire(l + _FD, bw)
                else:
                    @pl.when(s < n_super - 1)
                    def _():
                        fire(l + _FD, bw)
                drain_gather(b)
                if b + _FD < _NBUF:
                    @pl.when(s >= 1)
                    def _():
                        out_wait(tb)
                else:
                    out_wait(tb)
                transpose(b, tb)
                out_start(l, tb)
            return carry

        lax.fori_loop(0, n_super, super_iter, 0)

        for tb in range(2):
            out_wait(tb)

    return gather


def kernel(item_seq, item_emb_weight):
    b, l = item_seq.shape
    dim = item_emb_weight.shape[1]
    idx = item_seq.astype(jnp.int32)
    k = _build_gather(b, l, dim)(idx, item_emb_weight)
    k = k.reshape(l, dim // 8, _NW, 8, 128)
    return jnp.transpose(k, (2, 4, 0, 1, 3)).reshape(b, l, dim)


# speedup vs baseline: 1.2110x; 1.0684x over previous
import functools

import jax
import jax.numpy as jnp
from jax import lax
from jax.experimental import pallas as pl
from jax.experimental.pallas import tpu as pltpu
from jax.experimental.pallas import tpu_sc as plsc

_NC = 2
_NS = 16
_NW = _NC * _NS
_NBUF = 4
_FD = 2


@functools.cache
def _build_gather(b_total: int, l_total: int, dim: int):
    rows_per_w = b_total // _NW
    c0 = min(l_total, 128)
    c1 = l_total - c0
    n_chunks = 2 * rows_per_w
    n_super = n_chunks // _NBUF
    sz = {0: c0, 1: c1}
    off = {0: 0, 1: c0}
    mesh = plsc.VectorSubcoreMesh(
        core_axis_name="c", subcore_axis_name="s",
        num_cores=_NC, num_subcores=_NS,
    )

    @functools.partial(
        pl.kernel,
        out_type=jax.ShapeDtypeStruct((b_total, l_total, dim), jnp.float32),
        mesh=mesh,
        scratch_types=[
            pltpu.VMEM((rows_per_w, l_total), jnp.int32),
            pltpu.VMEM((_NBUF, c0, dim), jnp.float32),
        ]
        + [pltpu.SemaphoreType.DMA] * (2 * _NBUF),
        compiler_params=pltpu.CompilerParams(use_tc_tiling_on_sc=False),
    )
    def gather(idx_hbm, table_hbm, out_hbm, idx_v, rows_v, *sems):
        gs = sems[:_NBUF]
        os_ = sems[_NBUF:]
        wid = lax.axis_index("s") * _NC + lax.axis_index("c")
        base = wid * rows_per_w

        pltpu.sync_copy(idx_hbm.at[pl.ds(base, rows_per_w)], idx_v)

        def fire(c, b):
            p = b % 2
            pltpu.async_copy(
                table_hbm.at[idx_v.at[c // 2, pl.ds(off[p], sz[p])]],
                rows_v.at[b, pl.ds(0, sz[p])], gs[b])

        def drain_gather(b):
            p = b % 2
            pltpu.make_async_copy(
                table_hbm.at[idx_v.at[0, pl.ds(off[p], sz[p])]],
                rows_v.at[b, pl.ds(0, sz[p])], gs[b]).wait()

        def out_start(c, b):
            p = b % 2
            pltpu.async_copy(
                rows_v.at[b, pl.ds(0, sz[p])],
                out_hbm.at[base + c // 2, pl.ds(off[p], sz[p])], os_[b])

        def out_wait(b):
            p = b % 2
            pltpu.make_async_copy(
                rows_v.at[b, pl.ds(0, sz[p])],
                out_hbm.at[0, pl.ds(off[p], sz[p])], os_[b]).wait()

        for cp in range(_FD):
            fire(cp, cp)

        def super_iter(s, carry):
            for b in range(_NBUF):
                g = s * _NBUF + b
                bw = (b + _FD) % _NBUF
                if b + _FD < _NBUF:
                    @pl.when(s >= 1)
                    def _():
                        out_wait(bw)
                        fire(g + _FD, bw)
                    @pl.when(s == 0)
                    def _():
                        fire(g + _FD, bw)
                else:
                    out_wait(bw)
                    @pl.when(s < n_super - 1)
                    def _():
                        fire(g + _FD, bw)
                drain_gather(b)
                out_start(g, b)
            return carry

        lax.fori_loop(0, n_super, super_iter, 0)

        for j in range(_NBUF - _FD):
            out_wait((_FD + j) % _NBUF)

    return gather


def kernel(item_seq, item_emb_weight):
    b, l = item_seq.shape
    dim = item_emb_weight.shape[1]
    idx = item_seq.astype(jnp.int32)
    return _build_gather(b, l, dim)(idx, item_emb_weight)
